# baseline (speedup 1.0000x reference)
import functools

import jax
import jax.numpy as jnp
from jax import lax
from jax.experimental import pallas as pl
from jax.experimental.pallas import tpu as pltpu
from jax.experimental.pallas import tpu_sc as plsc

NC = 2
NS = 16
LANE = 16
DH = 32
SUB = 125
GPC = 8
HB = GPC // 2
CHUNK = SUB * GPC


def _segsum_sc(tbl_l, tbl_r, src2d, dst2d, val, base_l, base_r, scale):
    n = base_l.shape[0]
    e_total = val.shape[0]
    ept = e_total // NS
    n_chunks = ept // CHUNK
    assert ept * NS == e_total and n_chunks * CHUNK == ept
    assert n_chunks % 2 == 0
    fr = 200
    nblk = n // fr
    assert nblk * fr == n
    hbe = SUB * 2

    mesh = plsc.VectorSubcoreMesh(core_axis_name="c", subcore_axis_name="s",
                                  num_cores=NC, num_subcores=NS)
    out_t = (jax.ShapeDtypeStruct((n, DH), jnp.float32),
             jax.ShapeDtypeStruct((n, DH), jnp.float32))

    @functools.partial(
        pl.kernel, out_type=out_t, mesh=mesh,
        compiler_params=pltpu.CompilerParams(use_tc_tiling_on_sc=False,
                                             needs_layout_passes=False),
        scratch_types=[
            pltpu.VMEM_SHARED((n, DH), jnp.float32),
            pltpu.VMEM((GPC, SUB), jnp.int32),
            pltpu.VMEM((GPC, SUB), jnp.int32),
            pltpu.VMEM((GPC, SUB), jnp.int32),
            pltpu.VMEM((GPC, SUB), jnp.int32),
            pltpu.VMEM((CHUNK,), jnp.float32),
            pltpu.VMEM((CHUNK,), jnp.float32),
            pltpu.VMEM((SUB * 2, DH), jnp.float32),
            pltpu.VMEM((SUB * 2, DH), jnp.float32),
            pltpu.VMEM((fr, DH), jnp.float32),
            pltpu.SemaphoreType.DMA,
            pltpu.SemaphoreType.DMA,
            pltpu.SemaphoreType.DMA,
            pltpu.SemaphoreType.DMA,
        ],
    )
    def k(tl_h, tr_h, src_h, dst_h, val_h, bl_h, br_h, ol_h, or_h,
          accum, idx0, idx1, dst0, dst1, val0, val1, rows0, rows1, bbuf,
          gsem, ssem, lsem0, lsem1):
        c = lax.axis_index("c")
        s = lax.axis_index("s")

        def init(base_h):
            @pl.loop(s, nblk, step=NS)
            def _(blk):
                r0 = blk * fr
                pltpu.sync_copy(base_h.at[pl.ds(r0, fr)], bbuf)

                @plsc.parallel_loop(0, fr, unroll=4)
                def _(i):
                    bbuf[i, pl.ds(0, LANE)] = bbuf[i, pl.ds(0, LANE)] * scale
                    bbuf[i, pl.ds(LANE, LANE)] = bbuf[i, pl.ds(LANE, LANE)] * scale

                pltpu.sync_copy(bbuf, accum.at[pl.ds(r0, fr)])

        def edges(tbl_h):
            rows = (rows0, rows1)
            idxs = (idx0, idx1)
            dsts = (dst0, dst1)
            vals = (val0, val1)
            lsems = (lsem0, lsem1)

            def load_descs(cc, b, make):
                row0 = (s * n_chunks + cc) * GPC
                e0 = (s * n_chunks + cc) * CHUNK
                f = pltpu.make_async_copy if make else pltpu.async_copy
                return [f(src_h.at[pl.ds(row0, GPC)], idxs[b], lsems[b]),
                        f(dst_h.at[pl.ds(row0, GPC)], dsts[b], lsems[b]),
                        f(val_h.at[pl.ds(e0, CHUNK)], vals[b], lsems[b])]

            def fire_g(b, hbq, rb):
                return [pltpu.async_copy(tbl_h.at[idxs[b].at[2 * hbq + j]],
                                         rows[rb].at[pl.ds(j * SUB, SUB)], gsem)
                        for j in range(2)]

            def fire_a(b, hbq, rb):
                return []

            def scale(b, hbq, rb):
                @plsc.parallel_loop(0, hbe, unroll=4)
                def _(e):
                    vi = jnp.full((LANE,), hbq * hbe, jnp.int32) + e
                    v = plsc.load_gather(vals[b], [vi])
                    rb_ = rows[rb]
                    rb_[e, pl.ds(0, LANE)] = rb_[e, pl.ds(0, LANE)] * v
                    rb_[e, pl.ds(LANE, LANE)] = rb_[e, pl.ds(LANE, LANE)] * v

            load_descs(0, 0, make=False)

            @pl.loop(0, n_chunks // 2)
            def _(t):
                c_even = 2 * t
                for d in load_descs(c_even, 0, make=True):
                    d.wait()
                gd = {0: fire_g(0, 0, 0)}
                ld1 = load_descs(c_even + 1, 1, make=False)
                ad = {}
                for hb in range(8):
                    b, hbq = (0, hb) if hb < 4 else (1, hb - 4)
                    rb = hb % 2
                    for d in gd.pop(hb):
                        d.wait()
                    if hb >= 1:
                        for d in ad.pop(hb - 1):
                            d.wait()
                    if hb == 3:
                        for d in ld1:
                            d.wait()
                    if hb < 7:
                        nb, nhbq = (0, hb + 1) if hb + 1 < 4 else (1, hb - 3)
                        gd[hb + 1] = fire_g(nb, nhbq, (hb + 1) % 2)
                    scale(b, hbq, rb)
                    ad[hb] = fire_a(b, hbq, rb)

                @pl.when(t + 1 < n_chunks // 2)
                def _():
                    load_descs(c_even + 2, 0, make=False)

                for d in ad.pop(7):
                    d.wait()

        def flush(out_h):
            @pl.loop(s, nblk, step=NS)
            def _(blk):
                r0 = blk * fr
                pltpu.sync_copy(accum.at[pl.ds(r0, fr)], bbuf)
                pltpu.sync_copy(bbuf, out_h.at[pl.ds(r0, fr)])

        pl.when(c == 0)(lambda: init(bl_h))
        pl.when(c == 1)(lambda: init(br_h))
        plsc.subcore_barrier()
        pl.when(c == 0)(lambda: edges(tl_h))
        pl.when(c == 1)(lambda: edges(tr_h))
        plsc.subcore_barrier()
        pl.when(c == 0)(lambda: flush(ol_h))
        pl.when(c == 1)(lambda: flush(or_h))

    return k(tbl_l, tbl_r, src2d, dst2d, val, base_l, base_r)


BU = 1000


def _social_matmul(sl, sr, w):
    n = sl.shape[0]

    def body(sl_ref, sr_ref, w_ref, ol_ref, or_ref):
        wm = w_ref[...]
        y = (jnp.dot(sl_ref[...], wm[:DH, :], preferred_element_type=jnp.float32)
             + jnp.dot(sr_ref[...], wm[DH:, :], preferred_element_type=jnp.float32))
        ol_ref[...] = y[:, :DH]
        or_ref[...] = y[:, DH:]

    half = pl.BlockSpec((BU, DH), lambda i: (i, 0))
    return pl.pallas_call(
        body,
        grid=(n // BU,),
        in_specs=[half, half, pl.BlockSpec((2 * DH, 2 * DH), lambda i: (0, 0))],
        out_specs=[half, half],
        out_shape=[jax.ShapeDtypeStruct((n, DH), jnp.float32)] * 2,
    )(sl, sr, w)


def _ln(x, g, b):
    m = jnp.mean(x, axis=-1, keepdims=True)
    v = jnp.mean((x - m) ** 2, axis=-1, keepdims=True)
    return (x - m) / jnp.sqrt(v + 1e-5) * g + b


def _lrelu(x):
    return jnp.where(x >= 0, x, 0.01 * x)


def _user_head(u0, u1l, u1r, u2l, u2r, s2l, s2r, lw,
               ho_W, ho_b, ho_g, ho_be, mp_W, mp_b, mp_g, mp_be,
               ou_W1, ou_b1, ou_g1, ou_be1, ou_W2, ou_b2):
    n = u0.shape[0]

    def body(lw_ref, u0_ref, u1l_ref, u1r_ref, u2l_ref, u2r_ref,
             s2l_ref, s2r_ref, how_ref, hob_ref, hog_ref, hobe_ref,
             mpw_ref, mpb_ref, mpg_ref, mpbe_ref, ouw1_ref, oub1_ref,
             oug1_ref, oube1_ref, ouw2_ref, oub2_ref, out_ref):
        w0, w1, w2 = lw_ref[0], lw_ref[1], lw_ref[2]
        u1 = jnp.concatenate([u1l_ref[...], u1r_ref[...]], axis=1)
        u2 = jnp.concatenate([u2l_ref[...], u2r_ref[...]], axis=1)
        s2 = jnp.concatenate([s2l_ref[...], s2r_ref[...]], axis=1)
        ulgc = w0 * u0_ref[...] + w1 * u1 + w2 * u2
        ho = _lrelu(_ln(jnp.dot(s2, how_ref[...], preferred_element_type=jnp.float32)
                        + hob_ref[...], hog_ref[...], hobe_ref[...]))
        mpw = mpw_ref[...]
        fused = _lrelu(_ln(
            jnp.dot(ulgc, mpw[:64, :], preferred_element_type=jnp.float32)
            + jnp.dot(ho, mpw[64:, :], preferred_element_type=jnp.float32)
            + mpb_ref[...], mpg_ref[...], mpbe_ref[...]))
        ouw1 = ouw1_ref[...]
        h = _lrelu(_ln(
            jnp.dot(ulgc, ouw1[:64, :], preferred_element_type=jnp.float32)
            + jnp.dot(ho, ouw1[64:128, :], preferred_element_type=jnp.float32)
            + jnp.dot(fused, ouw1[128:, :], preferred_element_type=jnp.float32)
            + oub1_ref[...], oug1_ref[...], oube1_ref[...]))
        out_ref[...] = (jnp.dot(h, ouw2_ref[...], preferred_element_type=jnp.float32)
                        + oub2_ref[...])

    full = pl.BlockSpec((BU, 64), lambda i: (i, 0))
    half = pl.BlockSpec((BU, DH), lambda i: (i, 0))

    def whole(a):
        return pl.BlockSpec(a.shape, lambda i: (0,) * a.ndim)

    mats = [ho_W, ho_b, ho_g, ho_be, mp_W, mp_b, mp_g, mp_be,
            ou_W1, ou_b1, ou_g1, ou_be1, ou_W2, ou_b2]
    return pl.pallas_call(
        body,
        grid=(n // BU,),
        in_specs=[pl.BlockSpec(memory_space=pltpu.SMEM),
                  full, half, half, half, half, half, half]
                 + [whole(a) for a in mats],
        out_specs=full,
        out_shape=jax.ShapeDtypeStruct((n, 64), jnp.float32),
    )(lw, u0, u1l, u1r, u2l, u2r, s2l, s2r, *mats)


def _item_head(i0, i1l, i1r, i2l, i2r, lw, oi_W, oi_b, oi_g, oi_be):
    n = i0.shape[0]

    def body(lw_ref, i0_ref, i1l_ref, i1r_ref, i2l_ref, i2r_ref,
             oiw_ref, oib_ref, oig_ref, oibe_ref, out_ref):
        w0, w1, w2 = lw_ref[0], lw_ref[1], lw_ref[2]
        i1 = jnp.concatenate([i1l_ref[...], i1r_ref[...]], axis=1)
        i2 = jnp.concatenate([i2l_ref[...], i2r_ref[...]], axis=1)
        i0v = i0_ref[...]
        ilgc = w0 * i0v + w1 * i1 + w2 * i2
        oiw = oiw_ref[...]
        out_ref[...] = _lrelu(_ln(
            jnp.dot(ilgc, oiw[:64, :], preferred_element_type=jnp.float32)
            + jnp.dot(i0v, oiw[64:, :], preferred_element_type=jnp.float32)
            + oib_ref[...], oig_ref[...], oibe_ref[...]))

    full = pl.BlockSpec((BU, 64), lambda i: (i, 0))
    half = pl.BlockSpec((BU, DH), lambda i: (i, 0))

    def whole(a):
        return pl.BlockSpec(a.shape, lambda i: (0,) * a.ndim)

    mats = [oi_W, oi_b, oi_g, oi_be]
    return pl.pallas_call(
        body,
        grid=(n // BU,),
        in_specs=[pl.BlockSpec(memory_space=pltpu.SMEM),
                  full, half, half, half, half] + [whole(a) for a in mats],
        out_specs=full,
        out_shape=jax.ShapeDtypeStruct((n, 64), jnp.float32),
    )(lw, i0, i1l, i1r, i2l, i2r, *mats)


def kernel(ui_edge_index, ui_val, social_edge_index, social_val, user_embedding,
           item_embedding, social_W0, social_W1, layer_weights, mp_W, mp_b, mp_g,
           mp_be, ho_W, ho_b, ho_g, ho_be, ou_W1, ou_b1, ou_g1, ou_be1, ou_W2,
           ou_b2, oi_W, oi_b, oi_g, oi_be):
    f32 = jnp.float32
    u0 = user_embedding.astype(f32)
    i0 = item_embedding.astype(f32)

    u_idx = ui_edge_index[0].astype(jnp.int32)
    i_idx = ui_edge_index[1].astype(jnp.int32)
    s_row = social_edge_index[0].astype(jnp.int32)
    s_col = social_edge_index[1].astype(jnp.int32)
    e = u_idx.shape[0]
    es = s_row.shape[0]
    u_2d = u_idx.reshape(e // SUB, SUB)
    i_2d = i_idx.reshape(e // SUB, SUB)
    sr_2d = s_row.reshape(es // SUB, SUB)
    sc_2d = s_col.reshape(es // SUB, SUB)
    uival = ui_val.astype(f32)
    sval = social_val.astype(f32)

    u0l, u0r = u0[:, :DH], u0[:, DH:]
    i0l, i0r = i0[:, :DH], i0[:, DH:]

    u1l, u1r = _segsum_sc(i0l, i0r, i_2d, u_2d, uival, u0l, u0r, 0.1)
    t1l, t1r = _segsum_sc(u0l, u0r, u_2d, i_2d, uival, i0l, i0r, 0.1)
    u2l, u2r = _segsum_sc(t1l, t1r, i_2d, u_2d, uival, u0l, u0r, 0.1)
    t2l, t2r = _segsum_sc(u1l, u1r, u_2d, i_2d, uival, i0l, i0r, 0.1)

    w0l, w0r = _social_matmul(u0l, u0r, social_W0.astype(f32))
    s1l, s1r = _segsum_sc(w0l, w0r, sc_2d, sr_2d, sval, u0l, u0r, 1.0)
    w1l, w1r = _social_matmul(s1l, s1r, social_W1.astype(f32))
    s2l, s2r = _segsum_sc(w1l, w1r, sc_2d, sr_2d, sval, s1l, s1r, 1.0)

    lw = jax.nn.softmax(layer_weights[:3].astype(f32))

    def row(v):
        return v.astype(f32).reshape(1, -1)

    final_user = _user_head(
        u0, u1l, u1r, u2l, u2r, s2l, s2r, lw,
        ho_W.astype(f32), row(ho_b), row(ho_g), row(ho_be),
        mp_W.astype(f32), row(mp_b), row(mp_g), row(mp_be),
        ou_W1.astype(f32), row(ou_b1), row(ou_g1), row(ou_be1),
        ou_W2.astype(f32), row(ou_b2))
    final_item = _item_head(
        i0, t1l, t1r, t2l, t2r, lw,
        oi_W.astype(f32), row(oi_b), row(oi_g), row(oi_be))
    return (final_user, final_item)


# speedup vs baseline: 1.0501x; 1.0501x over previous
import functools

import jax
import jax.numpy as jnp
from jax import lax
from jax.experimental import pallas as pl
from jax.experimental.pallas import tpu as pltpu
from jax.experimental.pallas import tpu_sc as plsc

NC = 2
NS = 16
LANE = 16
DH = 32
SUB = 125
GPC = 8
HB = GPC // 2
CHUNK = SUB * GPC


def _segsum_sc(tbl_l, tbl_r, src2d, dst2d, val, base_l, base_r, scale):
    n = base_l.shape[0]
    e_total = val.shape[0]
    ept = e_total // NS
    n_chunks = ept // CHUNK
    assert ept * NS == e_total and n_chunks * CHUNK == ept
    assert n_chunks % 2 == 0
    fr = 200
    nblk = n // fr
    assert nblk * fr == n
    hbe = SUB * 2

    mesh = plsc.VectorSubcoreMesh(core_axis_name="c", subcore_axis_name="s",
                                  num_cores=NC, num_subcores=NS)
    out_t = (jax.ShapeDtypeStruct((n, DH), jnp.float32),
             jax.ShapeDtypeStruct((n, DH), jnp.float32))

    @functools.partial(
        pl.kernel, out_type=out_t, mesh=mesh,
        compiler_params=pltpu.CompilerParams(use_tc_tiling_on_sc=False,
                                             needs_layout_passes=False),
        scratch_types=[
            pltpu.VMEM_SHARED((n, DH), jnp.float32),
            pltpu.VMEM((GPC, SUB), jnp.int32),
            pltpu.VMEM((GPC, SUB), jnp.int32),
            pltpu.VMEM((GPC, SUB), jnp.int32),
            pltpu.VMEM((GPC, SUB), jnp.int32),
            pltpu.VMEM((CHUNK,), jnp.float32),
            pltpu.VMEM((CHUNK,), jnp.float32),
            pltpu.VMEM((SUB, DH), jnp.float32),
            pltpu.VMEM((SUB, DH), jnp.float32),
            pltpu.VMEM((SUB, DH), jnp.float32),
            pltpu.VMEM((SUB, DH), jnp.float32),
            pltpu.VMEM((fr, DH), jnp.float32),
            pltpu.SemaphoreType.DMA,
            pltpu.SemaphoreType.DMA,
            pltpu.SemaphoreType.DMA,
            pltpu.SemaphoreType.DMA,
        ],
    )
    def k(tl_h, tr_h, src_h, dst_h, val_h, bl_h, br_h, ol_h, or_h,
          accum, idx0, idx1, dst0, dst1, val0, val1, rows0, rows1, rows2,
          rows3, bbuf, gsem, ssem, lsem0, lsem1):
        c = lax.axis_index("c")
        s = lax.axis_index("s")

        def init(base_h):
            @pl.loop(s, nblk, step=NS)
            def _(blk):
                r0 = blk * fr
                pltpu.sync_copy(base_h.at[pl.ds(r0, fr)], bbuf)

                @plsc.parallel_loop(0, fr, unroll=4)
                def _(i):
                    bbuf[i, pl.ds(0, LANE)] = bbuf[i, pl.ds(0, LANE)] * scale
                    bbuf[i, pl.ds(LANE, LANE)] = bbuf[i, pl.ds(LANE, LANE)] * scale

                pltpu.sync_copy(bbuf, accum.at[pl.ds(r0, fr)])

        def edges(tbl_h):
            rows = (rows0, rows1, rows2, rows3)
            idxs = (idx0, idx1)
            dsts = (dst0, dst1)
            vals = (val0, val1)
            lsems = (lsem0, lsem1)
            nbuf = len(rows)
            pre = nbuf - 1
            nst = 2 * GPC

            def load_descs(cc, b, make):
                row0 = (s * n_chunks + cc) * GPC
                e0 = (s * n_chunks + cc) * CHUNK
                f = pltpu.make_async_copy if make else pltpu.async_copy
                return [f(src_h.at[pl.ds(row0, GPC)], idxs[b], lsems[b]),
                        f(dst_h.at[pl.ds(row0, GPC)], dsts[b], lsems[b]),
                        f(val_h.at[pl.ds(e0, CHUNK)], vals[b], lsems[b])]

            def fire_g(i):
                b, q = divmod(i, GPC)
                return pltpu.async_copy(tbl_h.at[idxs[b].at[q]],
                                        rows[i % nbuf], gsem)

            def fire_a(i):
                b, q = divmod(i, GPC)
                return pltpu.async_copy(rows[i % nbuf],
                                        accum.at[dsts[b].at[q]], ssem, add=True)

            def scale(i):
                b, q = divmod(i, GPC)
                rb_ = rows[i % nbuf]

                @plsc.parallel_loop(0, SUB, unroll=4)
                def _(e):
                    vi = jnp.full((LANE,), q * SUB, jnp.int32) + e
                    v = plsc.load_gather(vals[b], [vi])
                    rb_[e, pl.ds(0, LANE)] = rb_[e, pl.ds(0, LANE)] * v
                    rb_[e, pl.ds(LANE, LANE)] = rb_[e, pl.ds(LANE, LANE)] * v

            load_descs(0, 0, make=False)

            @pl.loop(0, n_chunks // 2)
            def _(t):
                c_even = 2 * t
                for d in load_descs(c_even, 0, make=True):
                    d.wait()
                gd = {i: fire_g(i) for i in range(pre)}
                ld1 = load_descs(c_even + 1, 1, make=False)
                ad = {}
                for i in range(nst):
                    j = i + pre
                    if i >= 1:
                        ad.pop(i - 1).wait()
                    if j < nst:
                        if j == GPC:
                            for d in ld1:
                                d.wait()
                        gd[j] = fire_g(j)
                    gd.pop(i).wait()
                    scale(i)
                    ad[i] = fire_a(i)

                @pl.when(t + 1 < n_chunks // 2)
                def _():
                    load_descs(c_even + 2, 0, make=False)

                ad.pop(nst - 1).wait()

        def flush(out_h):
            @pl.loop(s, nblk, step=NS)
            def _(blk):
                r0 = blk * fr
                pltpu.sync_copy(accum.at[pl.ds(r0, fr)], bbuf)
                pltpu.sync_copy(bbuf, out_h.at[pl.ds(r0, fr)])

        pl.when(c == 0)(lambda: init(bl_h))
        pl.when(c == 1)(lambda: init(br_h))
        plsc.subcore_barrier()
        pl.when(c == 0)(lambda: edges(tl_h))
        pl.when(c == 1)(lambda: edges(tr_h))
        plsc.subcore_barrier()
        pl.when(c == 0)(lambda: flush(ol_h))
        pl.when(c == 1)(lambda: flush(or_h))

    return k(tbl_l, tbl_r, src2d, dst2d, val, base_l, base_r)


BU = 1000


def _social_matmul(sl, sr, w):
    n = sl.shape[0]

    def body(sl_ref, sr_ref, w_ref, ol_ref, or_ref):
        wm = w_ref[...]
        y = (jnp.dot(sl_ref[...], wm[:DH, :], preferred_element_type=jnp.float32)
             + jnp.dot(sr_ref[...], wm[DH:, :], preferred_element_type=jnp.float32))
        ol_ref[...] = y[:, :DH]
        or_ref[...] = y[:, DH:]

    half = pl.BlockSpec((BU, DH), lambda i: (i, 0))
    return pl.pallas_call(
        body,
        grid=(n // BU,),
        in_specs=[half, half, pl.BlockSpec((2 * DH, 2 * DH), lambda i: (0, 0))],
        out_specs=[half, half],
        out_shape=[jax.ShapeDtypeStruct((n, DH), jnp.float32)] * 2,
    )(sl, sr, w)


def _ln(x, g, b):
    m = jnp.mean(x, axis=-1, keepdims=True)
    v = jnp.mean((x - m) ** 2, axis=-1, keepdims=True)
    return (x - m) / jnp.sqrt(v + 1e-5) * g + b


def _lrelu(x):
    return jnp.where(x >= 0, x, 0.01 * x)


def _user_head(u0, u1l, u1r, u2l, u2r, s2l, s2r, lw,
               ho_W, ho_b, ho_g, ho_be, mp_W, mp_b, mp_g, mp_be,
               ou_W1, ou_b1, ou_g1, ou_be1, ou_W2, ou_b2):
    n = u0.shape[0]

    def body(lw_ref, u0_ref, u1l_ref, u1r_ref, u2l_ref, u2r_ref,
             s2l_ref, s2r_ref, how_ref, hob_ref, hog_ref, hobe_ref,
             mpw_ref, mpb_ref, mpg_ref, mpbe_ref, ouw1_ref, oub1_ref,
             oug1_ref, oube1_ref, ouw2_ref, oub2_ref, out_ref):
        w0, w1, w2 = lw_ref[0], lw_ref[1], lw_ref[2]
        u1 = jnp.concatenate([u1l_ref[...], u1r_ref[...]], axis=1)
        u2 = jnp.concatenate([u2l_ref[...], u2r_ref[...]], axis=1)
        s2 = jnp.concatenate([s2l_ref[...], s2r_ref[...]], axis=1)
        ulgc = w0 * u0_ref[...] + w1 * u1 + w2 * u2
        ho = _lrelu(_ln(jnp.dot(s2, how_ref[...], preferred_element_type=jnp.float32)
                        + hob_ref[...], hog_ref[...], hobe_ref[...]))
        mpw = mpw_ref[...]
        fused = _lrelu(_ln(
            jnp.dot(ulgc, mpw[:64, :], preferred_element_type=jnp.float32)
            + jnp.dot(ho, mpw[64:, :], preferred_element_type=jnp.float32)
            + mpb_ref[...], mpg_ref[...], mpbe_ref[...]))
        ouw1 = ouw1_ref[...]
        h = _lrelu(_ln(
            jnp.dot(ulgc, ouw1[:64, :], preferred_element_type=jnp.float32)
            + jnp.dot(ho, ouw1[64:128, :], preferred_element_type=jnp.float32)
            + jnp.dot(fused, ouw1[128:, :], preferred_element_type=jnp.float32)
            + oub1_ref[...], oug1_ref[...], oube1_ref[...]))
        out_ref[...] = (jnp.dot(h, ouw2_ref[...], preferred_element_type=jnp.float32)
                        + oub2_ref[...])

    full = pl.BlockSpec((BU, 64), lambda i: (i, 0))
    half = pl.BlockSpec((BU, DH), lambda i: (i, 0))

    def whole(a):
        return pl.BlockSpec(a.shape, lambda i: (0,) * a.ndim)

    mats = [ho_W, ho_b, ho_g, ho_be, mp_W, mp_b, mp_g, mp_be,
            ou_W1, ou_b1, ou_g1, ou_be1, ou_W2, ou_b2]
    return pl.pallas_call(
        body,
        grid=(n // BU,),
        in_specs=[pl.BlockSpec(memory_space=pltpu.SMEM),
                  full, half, half, half, half, half, half]
                 + [whole(a) for a in mats],
        out_specs=full,
        out_shape=jax.ShapeDtypeStruct((n, 64), jnp.float32),
    )(lw, u0, u1l, u1r, u2l, u2r, s2l, s2r, *mats)


def _item_head(i0, i1l, i1r, i2l, i2r, lw, oi_W, oi_b, oi_g, oi_be):
    n = i0.shape[0]

    def body(lw_ref, i0_ref, i1l_ref, i1r_ref, i2l_ref, i2r_ref,
             oiw_ref, oib_ref, oig_ref, oibe_ref, out_ref):
        w0, w1, w2 = lw_ref[0], lw_ref[1], lw_ref[2]
        i1 = jnp.concatenate([i1l_ref[...], i1r_ref[...]], axis=1)
        i2 = jnp.concatenate([i2l_ref[...], i2r_ref[...]], axis=1)
        i0v = i0_ref[...]
        ilgc = w0 * i0v + w1 * i1 + w2 * i2
        oiw = oiw_ref[...]
        out_ref[...] = _lrelu(_ln(
            jnp.dot(ilgc, oiw[:64, :], preferred_element_type=jnp.float32)
            + jnp.dot(i0v, oiw[64:, :], preferred_element_type=jnp.float32)
            + oib_ref[...], oig_ref[...], oibe_ref[...]))

    full = pl.BlockSpec((BU, 64), lambda i: (i, 0))
    half = pl.BlockSpec((BU, DH), lambda i: (i, 0))

    def whole(a):
        return pl.BlockSpec(a.shape, lambda i: (0,) * a.ndim)

    mats = [oi_W, oi_b, oi_g, oi_be]
    return pl.pallas_call(
        body,
        grid=(n // BU,),
        in_specs=[pl.BlockSpec(memory_space=pltpu.SMEM),
                  full, half, half, half, half] + [whole(a) for a in mats],
        out_specs=full,
        out_shape=jax.ShapeDtypeStruct((n, 64), jnp.float32),
    )(lw, i0, i1l, i1r, i2l, i2r, *mats)


def kernel(ui_edge_index, ui_val, social_edge_index, social_val, user_embedding,
           item_embedding, social_W0, social_W1, layer_weights, mp_W, mp_b, mp_g,
           mp_be, ho_W, ho_b, ho_g, ho_be, ou_W1, ou_b1, ou_g1, ou_be1, ou_W2,
           ou_b2, oi_W, oi_b, oi_g, oi_be):
    f32 = jnp.float32
    u0 = user_embedding.astype(f32)
    i0 = item_embedding.astype(f32)

    u_idx = ui_edge_index[0].astype(jnp.int32)
    i_idx = ui_edge_index[1].astype(jnp.int32)
    s_row = social_edge_index[0].astype(jnp.int32)
    s_col = social_edge_index[1].astype(jnp.int32)
    e = u_idx.shape[0]
    es = s_row.shape[0]
    u_2d = u_idx.reshape(e // SUB, SUB)
    i_2d = i_idx.reshape(e // SUB, SUB)
    sr_2d = s_row.reshape(es // SUB, SUB)
    sc_2d = s_col.reshape(es // SUB, SUB)
    uival = ui_val.astype(f32)
    sval = social_val.astype(f32)

    u0l, u0r = u0[:, :DH], u0[:, DH:]
    i0l, i0r = i0[:, :DH], i0[:, DH:]

    u1l, u1r = _segsum_sc(i0l, i0r, i_2d, u_2d, uival, u0l, u0r, 0.1)
    t1l, t1r = _segsum_sc(u0l, u0r, u_2d, i_2d, uival, i0l, i0r, 0.1)
    u2l, u2r = _segsum_sc(t1l, t1r, i_2d, u_2d, uival, u0l, u0r, 0.1)
    t2l, t2r = _segsum_sc(u1l, u1r, u_2d, i_2d, uival, i0l, i0r, 0.1)

    w0l, w0r = _social_matmul(u0l, u0r, social_W0.astype(f32))
    s1l, s1r = _segsum_sc(w0l, w0r, sc_2d, sr_2d, sval, u0l, u0r, 1.0)
    w1l, w1r = _social_matmul(s1l, s1r, social_W1.astype(f32))
    s2l, s2r = _segsum_sc(w1l, w1r, sc_2d, sr_2d, sval, s1l, s1r, 1.0)

    lw = jax.nn.softmax(layer_weights[:3].astype(f32))

    def row(v):
        return v.astype(f32).reshape(1, -1)

    final_user = _user_head(
        u0, u1l, u1r, u2l, u2r, s2l, s2r, lw,
        ho_W.astype(f32), row(ho_b), row(ho_g), row(ho_be),
        mp_W.astype(f32), row(mp_b), row(mp_g), row(mp_be),
        ou_W1.astype(f32), row(ou_b1), row(ou_g1), row(ou_be1),
        ou_W2.astype(f32), row(ou_b2))
    final_item = _item_head(
        i0, t1l, t1r, t2l, t2r, lw,
        oi_W.astype(f32), row(oi_b), row(oi_g), row(oi_be))
    return (final_user, final_item)
